# Initial kernel scaffold; baseline (speedup 1.0000x reference)
import jax
import jax.numpy as jnp
from jax.experimental import pallas as pl


def kernel(indices, embedding):
    raise NotImplementedError("write your pallas kernel here")


# speedup vs baseline: 1.1563x; 1.1563x over previous
import functools

import jax
import jax.numpy as jnp
from jax import lax
from jax.experimental import pallas as pl
from jax.experimental.pallas import tpu as pltpu
from jax.experimental.pallas import tpu_sc as plsc

TOT = 128
D = 4096
BATCH = 128
B = BATCH * TOT

_info = plsc.get_sparse_core_info()
NC, NS = _info.num_cores, _info.num_subcores
NW = NC * NS
B_PER_W = B // NW
C = 8
G = B_PER_W // C


def _body(idx_hbm, table_hbm, out_hbm, idx_v, buf, gsem):
    wid = lax.axis_index("s") * NC + lax.axis_index("c")
    pltpu.sync_copy(idx_hbm.at[wid], idx_v)

    def step(g, carry):
        pltpu.async_copy(table_hbm.at[idx_v.at[g]], buf, gsem).wait()
        pltpu.sync_copy(buf, out_hbm.at[pl.ds(wid * B_PER_W + g * C, C)])
        return carry

    lax.fori_loop(0, G, step, 0)


_gather = pl.kernel(
    _body,
    out_type=jax.ShapeDtypeStruct((B, D), jnp.float32),
    mesh=plsc.VectorSubcoreMesh(core_axis_name="c", subcore_axis_name="s"),
    scratch_types=[
        pltpu.VMEM((G, C), jnp.int32),
        pltpu.VMEM((C, D), jnp.float32),
        pltpu.SemaphoreType.DMA,
    ],
)


def kernel(indices, embedding):
    idx = indices.astype(jnp.int32).reshape(NW, G, C)
    out = _gather(idx, embedding)
    return out.reshape(BATCH, TOT, D)


# speedup vs baseline: 1.2730x; 1.1009x over previous
import functools

import jax
import jax.numpy as jnp
from jax import lax
from jax.experimental import pallas as pl
from jax.experimental.pallas import tpu as pltpu
from jax.experimental.pallas import tpu_sc as plsc

TOT = 128
D = 4096
BATCH = 128
B = BATCH * TOT

_info = plsc.get_sparse_core_info()
NC, NS = _info.num_cores, _info.num_subcores
NW = NC * NS
B_PER_W = B // NW
C = 8
G = B_PER_W // C


def _body(idx_hbm, table_hbm, out_hbm, idx_v, buf0, buf1, g0, g1, s0, s1):
    wid = lax.axis_index("s") * NC + lax.axis_index("c")
    base = wid * B_PER_W
    pltpu.sync_copy(idx_hbm.at[wid], idx_v)

    def fire_gather(g, buf, sem):
        pltpu.async_copy(table_hbm.at[idx_v.at[g]], buf, sem)

    def wait_gather(g, buf, sem):
        pltpu.make_async_copy(table_hbm.at[idx_v.at[g]], buf, sem).wait()

    def fire_scatter(g, buf, sem):
        pltpu.async_copy(buf, out_hbm.at[pl.ds(base + g * C, C)], sem)

    def wait_scatter(g, buf, sem):
        pltpu.make_async_copy(buf, out_hbm.at[pl.ds(base + g * C, C)], sem).wait()

    fire_gather(0, buf0, g0)

    def step(h, carry):
        a = 2 * h
        b = a + 1

        @pl.when(h >= 1)
        def _():
            wait_scatter(b - 2, buf1, s1)

        fire_gather(b, buf1, g1)
        wait_gather(a, buf0, g0)
        fire_scatter(a, buf0, s0)

        wait_scatter(a, buf0, s0)

        @pl.when(b + 1 < G)
        def _():
            fire_gather(b + 1, buf0, g0)

        wait_gather(b, buf1, g1)
        fire_scatter(b, buf1, s1)
        return carry

    lax.fori_loop(0, G // 2, step, 0)
    wait_scatter(G - 1, buf1, s1)


_gather = pl.kernel(
    _body,
    out_type=jax.ShapeDtypeStruct((B, D), jnp.float32),
    mesh=plsc.VectorSubcoreMesh(core_axis_name="c", subcore_axis_name="s"),
    scratch_types=[
        pltpu.VMEM((G, C), jnp.int32),
        pltpu.VMEM((C, D), jnp.float32),
        pltpu.VMEM((C, D), jnp.float32),
        pltpu.SemaphoreType.DMA,
        pltpu.SemaphoreType.DMA,
        pltpu.SemaphoreType.DMA,
        pltpu.SemaphoreType.DMA,
    ],
)


def kernel(indices, embedding):
    idx = indices.astype(jnp.int32).reshape(NW, G, C)
    out = _gather(idx, embedding)
    return out.reshape(BATCH, TOT, D)


# speedup vs baseline: 1.8993x; 1.4920x over previous
import jax
import jax.numpy as jnp
from jax import lax
from jax.experimental import pallas as pl
from jax.experimental.pallas import tpu as pltpu
from jax.experimental.pallas import tpu_sc as plsc

TOT = 128
D = 4096
BATCH = 128
B = BATCH * TOT

_info = plsc.get_sparse_core_info()
NC, NS = _info.num_cores, _info.num_subcores
NW = NC * NS
B_PER_W = B // NW
K = 16
NB = B_PER_W // K


def _body(idx_hbm, table_hbm, out_hbm, idx_v, table_sp, sem):
    sid = lax.axis_index("s")
    wid = sid * NC + lax.axis_index("c")
    base = wid * B_PER_W
    pltpu.sync_copy(idx_hbm.at[wid], idx_v)
    rpw = TOT // NS
    pltpu.sync_copy(table_hbm.at[pl.ds(sid * rpw, rpw)],
                    table_sp.at[pl.ds(sid * rpw, rpw)])
    plsc.subcore_barrier()

    def fire(g):
        vec = idx_v[pl.ds(g * K, K)]
        for jj in range(K):
            off = vec[jj]
            pltpu.async_copy(table_sp.at[pl.ds(off, 1)],
                             out_hbm.at[pl.ds(base + g * K + jj, 1)], sem)

    def drain(g):
        for jj in range(K):
            pltpu.make_async_copy(table_sp.at[pl.ds(0, 1)],
                                  out_hbm.at[pl.ds(base, 1)], sem).wait()

    fire(0)

    def step(g, carry):
        @pl.when(g + 1 < NB)
        def _():
            fire(g + 1)

        drain(g)
        return carry

    lax.fori_loop(0, NB, step, 0)


_gather = pl.kernel(
    _body,
    out_type=jax.ShapeDtypeStruct((B, D), jnp.float32),
    mesh=plsc.VectorSubcoreMesh(core_axis_name="c", subcore_axis_name="s"),
    scratch_types=[
        pltpu.VMEM((B_PER_W,), jnp.int32),
        pltpu.VMEM_SHARED((TOT, D), jnp.float32),
        pltpu.SemaphoreType.DMA,
    ],
)


def kernel(indices, embedding):
    idx = indices.astype(jnp.int32).reshape(NW, B_PER_W)
    out = _gather(idx, embedding)
    return out.reshape(BATCH, TOT, D)
